# baseline (speedup 1.0000x reference)
import functools

import jax
import jax.numpy as jnp
from jax import lax
from jax.experimental import pallas as pl
from jax.experimental.pallas import tpu as pltpu
from jax.experimental.pallas import tpu_sc as plsc

N = 4096
TOTAL = N * (N + 1) // 2
NC = 2
NS = 16
NW = NC * NS
PIECE = 1024
PIECE2 = 8192
SR = 128

BOUNDS = (0, 512, 1024, 2048, 2944, 3584, 4096)
G = len(BOUNDS) - 1


def _pad_w(hi):
    return ((hi + PIECE - 1) // PIECE) * PIECE


def _tri(i):
    return i * (i + 1) // 2


def _chunk_words(g):
    lo, hi = BOUNDS[g], BOUNDS[g + 1]
    cnt = (hi - lo) // NW
    ch = PIECE2
    while True:
        ok = True
        for w in range(NW):
            a = lo + w * cnt
            base = min(_tri(a) // 8 * 8, TOTAL - ch)
            mx = max(_tri(i) + -(-(i + 1) // 64) * 64 for i in range(a, a + cnt))
            if base < 0 or mx - base > ch:
                ok = False
                break
        if ok:
            return ch
        ch += PIECE2


def _densify_body(lo, hi, ch, ew_hbm, l_hbm, chunk, row0, row1,
                  sem_in, sem_o0, sem_o1):
    wid = lax.axis_index("s") * NC + lax.axis_index("c")
    iota16 = lax.iota(jnp.int32, 16)
    rows = (row0, row1)
    sems_o = (sem_o0, sem_o1)
    cnt = (hi - lo) // NW
    wpad = _pad_w(hi)
    a = lo + wid * cnt
    ta = a * (a + 1) // 2
    base = jnp.minimum(ta - lax.rem(ta, 8), TOTAL - ch)
    base = pl.multiple_of(base, 8)
    bcap = a + cnt
    need = (bcap * (bcap + 1)) // 2 - base

    np_pieces = ch // PIECE2
    for phase in (0, 1):
        for q in range(np_pieces):
            @pl.when(PIECE2 * q < need)
            def _():
                d = pltpu.make_async_copy(
                    ew_hbm.at[pl.ds(base + PIECE2 * q, PIECE2)],
                    chunk.at[pl.ds(PIECE2 * q, PIECE2)],
                    sem_in,
                )
                d.start() if phase == 0 else d.wait()

    def do_row(k, p, drain_pred):
        i = a + k
        off = (i * (i + 1)) // 2 - base

        @pl.when(drain_pred)
        def _():
            pltpu.make_async_copy(rows[p], l_hbm.at[i - 2 - lo], sems_o[p]).wait()

        n64 = ((i + 64) // 64) * 64

        @plsc.parallel_loop(0, n64, step=64, unroll=4)
        def _(q):
            for u in range(4):
                v = plsc.load_gather(chunk, [q + off + 16 * u + iota16])
                rows[p][pl.ds(q + 16 * u, 16)] = v
        pltpu.make_async_copy(rows[p], l_hbm.at[i - lo], sems_o[p]).start()

    def row_pair(k2, _):
        k = 2 * k2
        do_row(k, 0, k >= 2)
        do_row(k + 1, 1, k + 1 >= 2)
        return 0

    lax.fori_loop(0, cnt // 2, row_pair, 0)

    for k in (cnt - 2, cnt - 1):
        i = a + k
        pltpu.make_async_copy(rows[k % 2], l_hbm.at[i - lo], sems_o[k % 2]).wait()


def _densify_group(g, edge_weight):
    lo, hi = BOUNDS[g], BOUNDS[g + 1]
    ch = _chunk_words(g)
    mesh = plsc.VectorSubcoreMesh(
        core_axis_name="c", subcore_axis_name="s", num_cores=NC, num_subcores=NS
    )
    return pl.kernel(
        functools.partial(_densify_body, lo, hi, ch),
        out_type=jax.ShapeDtypeStruct((hi - lo, _pad_w(hi)), jnp.float32),
        mesh=mesh,
        compiler_params=pltpu.CompilerParams(needs_layout_passes=False),
        scratch_types=[
            pltpu.VMEM((ch,), jnp.float32),
            pltpu.VMEM((_pad_w(hi),), jnp.float32),
            pltpu.VMEM((_pad_w(hi),), jnp.float32),
            pltpu.SemaphoreType.DMA,
            pltpu.SemaphoreType.DMA,
            pltpu.SemaphoreType.DMA,
        ],
    )(edge_weight)


def _sym_body(g, lr_ref, *rest):
    lc_refs = rest[: G - g]
    m_ref = rest[G - g]
    o_ref = rest[-1]
    lo, hi = BOUNDS[g], BOUNDS[g + 1]
    s_glob = lo // SR + pl.program_id(0)
    lr = lr_ref[...]
    m = m_ref[...]
    pieces = []
    for h in range(G):
        lo_h, hi_h = BOUNDS[h], BOUNDS[h + 1]
        w = hi_h - lo_h
        mh = m[:, lo_h:hi_h]
        if h < g:
            pieces.append(lr[:, lo_h:hi_h] * mh)
        elif h == g:
            lct = lc_refs[0][...].T
            rr = SR * s_glob + lax.broadcasted_iota(jnp.int32, (SR, w), 0)
            cc = lo_h + lax.broadcasted_iota(jnp.int32, (SR, w), 1)
            pieces.append(jnp.where(cc <= rr, lr[:, lo_h:hi_h], lct) * mh)
        else:
            pieces.append(lc_refs[h - g][...].T * mh)
    val = jnp.concatenate(pieces, axis=1)
    o_ref[...] = val.reshape(SR * N)


def _sym_group(g, l_groups, mask, out_prev):
    lo, hi = BOUNDS[g], BOUNDS[g + 1]
    n_stripes = (hi - lo) // SR
    sb = lo // SR
    in_specs = [pl.BlockSpec((SR, hi), lambda s: (s, 0))]
    for h in range(g, G):
        rows_h = BOUNDS[h + 1] - BOUNDS[h]
        in_specs.append(
            pl.BlockSpec((rows_h, SR), lambda s, _sb=sb: (0, _sb + s))
        )
    in_specs.append(pl.BlockSpec((SR, N), lambda s, _sb=sb: (_sb + s, 0)))
    args = [l_groups[g], *l_groups[g:], mask]
    aliases = {}
    if out_prev is not None:
        in_specs.append(pl.BlockSpec(memory_space=pl.ANY))
        args.append(out_prev)
        aliases = {len(in_specs) - 1: 0}
    return pl.pallas_call(
        functools.partial(_sym_body, g),
        grid=(n_stripes,),
        in_specs=in_specs,
        out_specs=pl.BlockSpec((SR * N,), lambda s, _sb=sb: (_sb + s,)),
        out_shape=jax.ShapeDtypeStruct((N * N,), jnp.float32),
        input_output_aliases=aliases,
    )(*args)


@jax.jit
def kernel(mask, edge_weight, xs, ys):
    l_groups = [None] * G
    out = None
    for g in range(G - 1, -1, -1):
        l_groups[g] = _densify_group(g, edge_weight)
        out = _sym_group(g, l_groups, mask, out)
    return out


# speedup vs baseline: 1.0078x; 1.0078x over previous
import functools

import jax
import jax.numpy as jnp
from jax import lax
from jax.experimental import pallas as pl
from jax.experimental.pallas import tpu as pltpu
from jax.experimental.pallas import tpu_sc as plsc

N = 4096
TOTAL = N * (N + 1) // 2
NC = 2
NS = 16
NW = NC * NS
PIECE = 1024
PIECE2 = 8192
SR = 128

BOUNDS = (0, 1024, 2048, 2944, 3584, 4096)
G = len(BOUNDS) - 1


def _pad_w(hi):
    return ((hi + PIECE - 1) // PIECE) * PIECE


def _tri(i):
    return i * (i + 1) // 2


def _chunk_words(g):
    lo, hi = BOUNDS[g], BOUNDS[g + 1]
    cnt = (hi - lo) // NW
    ch = PIECE2
    while True:
        ok = True
        for w in range(NW):
            a = lo + w * cnt
            base = min(_tri(a) // 8 * 8, TOTAL - ch)
            mx = max(_tri(i) + -(-(i + 1) // 64) * 64 for i in range(a, a + cnt))
            if base < 0 or mx - base > ch:
                ok = False
                break
        if ok:
            return ch
        ch += PIECE2


def _densify_body(lo, hi, ch, ew_hbm, l_hbm, chunk, row0, row1,
                  sem_in, sem_o0, sem_o1):
    wid = lax.axis_index("s") * NC + lax.axis_index("c")
    iota16 = lax.iota(jnp.int32, 16)
    rows = (row0, row1)
    sems_o = (sem_o0, sem_o1)
    cnt = (hi - lo) // NW
    wpad = _pad_w(hi)
    a = lo + wid * cnt
    ta = a * (a + 1) // 2
    base = jnp.minimum(ta - lax.rem(ta, 8), TOTAL - ch)
    base = pl.multiple_of(base, 8)
    bcap = a + cnt
    need = (bcap * (bcap + 1)) // 2 - base

    np_pieces = ch // PIECE2
    for phase in (0, 1):
        for q in range(np_pieces):
            @pl.when(PIECE2 * q < need)
            def _():
                d = pltpu.make_async_copy(
                    ew_hbm.at[pl.ds(base + PIECE2 * q, PIECE2)],
                    chunk.at[pl.ds(PIECE2 * q, PIECE2)],
                    sem_in,
                )
                d.start() if phase == 0 else d.wait()

    def do_row(k, p, drain_pred):
        i = a + k
        off = (i * (i + 1)) // 2 - base

        @pl.when(drain_pred)
        def _():
            pltpu.make_async_copy(rows[p], l_hbm.at[i - 2 - lo], sems_o[p]).wait()

        n64 = ((i + 64) // 64) * 64

        @plsc.parallel_loop(0, n64, step=64, unroll=4)
        def _(q):
            for u in range(4):
                v = plsc.load_gather(chunk, [q + off + 16 * u + iota16])
                rows[p][pl.ds(q + 16 * u, 16)] = v
        pltpu.make_async_copy(rows[p], l_hbm.at[i - lo], sems_o[p]).start()

    def row_pair(k2, _):
        k = 2 * k2
        do_row(k, 0, k >= 2)
        do_row(k + 1, 1, k + 1 >= 2)
        return 0

    lax.fori_loop(0, cnt // 2, row_pair, 0)

    for k in (cnt - 2, cnt - 1):
        i = a + k
        pltpu.make_async_copy(rows[k % 2], l_hbm.at[i - lo], sems_o[k % 2]).wait()


def _densify_group(g, edge_weight):
    lo, hi = BOUNDS[g], BOUNDS[g + 1]
    ch = _chunk_words(g)
    mesh = plsc.VectorSubcoreMesh(
        core_axis_name="c", subcore_axis_name="s", num_cores=NC, num_subcores=NS
    )
    return pl.kernel(
        functools.partial(_densify_body, lo, hi, ch),
        out_type=jax.ShapeDtypeStruct((hi - lo, _pad_w(hi)), jnp.float32),
        mesh=mesh,
        compiler_params=pltpu.CompilerParams(needs_layout_passes=False),
        scratch_types=[
            pltpu.VMEM((ch,), jnp.float32),
            pltpu.VMEM((_pad_w(hi),), jnp.float32),
            pltpu.VMEM((_pad_w(hi),), jnp.float32),
            pltpu.SemaphoreType.DMA,
            pltpu.SemaphoreType.DMA,
            pltpu.SemaphoreType.DMA,
        ],
    )(edge_weight)


def _sym_body(g, lr_ref, *rest):
    lc_refs = rest[: G - g]
    m_ref = rest[G - g]
    o_ref = rest[-1]
    lo, hi = BOUNDS[g], BOUNDS[g + 1]
    s_glob = lo // SR + pl.program_id(0)
    lr = lr_ref[...]
    m = m_ref[...]
    pieces = []
    for h in range(G):
        lo_h, hi_h = BOUNDS[h], BOUNDS[h + 1]
        w = hi_h - lo_h
        mh = m[:, lo_h:hi_h]
        if h < g:
            pieces.append(lr[:, lo_h:hi_h] * mh)
        elif h == g:
            lct = lc_refs[0][...].T
            rr = SR * s_glob + lax.broadcasted_iota(jnp.int32, (SR, w), 0)
            cc = lo_h + lax.broadcasted_iota(jnp.int32, (SR, w), 1)
            pieces.append(jnp.where(cc <= rr, lr[:, lo_h:hi_h], lct) * mh)
        else:
            pieces.append(lc_refs[h - g][...].T * mh)
    val = jnp.concatenate(pieces, axis=1)
    o_ref[...] = val.reshape(SR * N)


def _sym_group(g, l_groups, mask, out_prev):
    lo, hi = BOUNDS[g], BOUNDS[g + 1]
    n_stripes = (hi - lo) // SR
    sb = lo // SR
    in_specs = [pl.BlockSpec((SR, hi), lambda s: (s, 0))]
    for h in range(g, G):
        rows_h = BOUNDS[h + 1] - BOUNDS[h]
        in_specs.append(
            pl.BlockSpec((rows_h, SR), lambda s, _sb=sb: (0, _sb + s))
        )
    in_specs.append(pl.BlockSpec((SR, N), lambda s, _sb=sb: (_sb + s, 0)))
    args = [l_groups[g], *l_groups[g:], mask]
    aliases = {}
    if out_prev is not None:
        in_specs.append(pl.BlockSpec(memory_space=pl.ANY))
        args.append(out_prev)
        aliases = {len(in_specs) - 1: 0}
    return pl.pallas_call(
        functools.partial(_sym_body, g),
        grid=(n_stripes,),
        in_specs=in_specs,
        out_specs=pl.BlockSpec((SR * N,), lambda s, _sb=sb: (_sb + s,)),
        out_shape=jax.ShapeDtypeStruct((N * N,), jnp.float32),
        input_output_aliases=aliases,
    )(*args)


@jax.jit
def kernel(mask, edge_weight, xs, ys):
    l_groups = [None] * G
    out = None
    for g in range(G - 1, -1, -1):
        l_groups[g] = _densify_group(g, edge_weight)
    for g in range(G - 1, -1, -1):
        out = _sym_group(g, l_groups, mask, out)
    return out
